# Initial kernel scaffold; baseline (speedup 1.0000x reference)
import jax
import jax.numpy as jnp
from jax.experimental import pallas as pl


def kernel(d, p, n, sdf_W1, sdf_b1, sdf_W2, sdf_b2, c1w0, c1a0, c1w1, c1a1, c2w0, c2a0, c2w1, c2a1, c3w0, c3a0, c4w, c4a, l1w, l1a, l2w, l2a, l3w, l3a, l4w, l4a):
    raise NotImplementedError("write your pallas kernel here")


# speedup vs baseline: 17.5812x; 17.5812x over previous
import functools

import jax
import jax.numpy as jnp
from jax import lax
from jax.experimental import pallas as pl
from jax.experimental.pallas import tpu as pltpu
from jax.experimental.pallas import tpu_sc as plsc

_K = 10
_N = 4096
_B = 2
_INTERPRET = False


def _prelu(x, a):
    return jnp.where(x >= 0, x, a * x)


_RB_KNN = 256


def _knn_body(rows_ref, all_ref, sqc_ref, sqr_ref, idx_ref):
    b = pl.program_id(0)
    xr = rows_ref[0]
    xa = all_ref[0]
    cross = lax.dot_general(xr, xa, (((1,), (1,)), ((), ())))
    d2 = sqc_ref[0] + sqr_ref[0] - 2.0 * cross
    cols = lax.broadcasted_iota(jnp.int32, d2.shape, 1)
    picks = []
    for _ in range(_K):
        m = jnp.min(d2, axis=1, keepdims=True)
        cand = jnp.where(d2 <= m, cols, _N)
        pj = jnp.min(cand, axis=1, keepdims=True)
        picks.append(pj)
        d2 = jnp.where(cols == pj, jnp.inf, d2)
    idx_ref[0] = jnp.concatenate(picks, axis=1) + b * _N


def _knn_call(feat):
    c = feat.shape[-1]
    sq = jnp.sum(feat * feat, axis=-1)
    sqc = sq[:, :, None]
    sqr = sq[:, None, :]
    grid = (_B, _N // _RB_KNN)
    return pl.pallas_call(
        _knn_body,
        grid=grid,
        in_specs=[
            pl.BlockSpec((1, _RB_KNN, c), lambda b, r: (b, r, 0)),
            pl.BlockSpec((1, _N, c), lambda b, r: (b, 0, 0)),
            pl.BlockSpec((1, _RB_KNN, 1), lambda b, r: (b, r, 0)),
            pl.BlockSpec((1, 1, _N), lambda b, r: (b, 0, 0)),
        ],
        out_specs=pl.BlockSpec((1, _RB_KNN, _K), lambda b, r: (b, r, 0)),
        out_shape=jax.ShapeDtypeStruct((_B, _N, _K), jnp.int32),
        interpret=_INTERPRET,
    )(feat, feat, sqc, sqr)


def _sc_gather(table2, idxg):
    c = table2.shape[-1]
    nw, _, per_w = idxg.shape
    m = per_w // 128
    ch = 4
    mesh = plsc.VectorSubcoreMesh(core_axis_name="c", subcore_axis_name="s")

    @functools.partial(
        pl.kernel,
        out_type=jax.ShapeDtypeStruct((nw * m, 128, c), jnp.float32),
        mesh=mesh,
        scratch_types=[
            pltpu.VMEM((1, per_w), jnp.int32),
            pltpu.VMEM((ch, 128, c), jnp.float32),
            pltpu.SemaphoreType.DMA,
        ],
        compiler_params=pltpu.CompilerParams(use_tc_tiling_on_sc=False),
    )
    def gk(table_hbm, idx_hbm, out_hbm, idx_v, rows_v, sem):
        wid = lax.axis_index("s") * 2 + lax.axis_index("c")
        ibase = wid * m
        pltpu.sync_copy(idx_hbm.at[wid], idx_v)
        for blk in range(m // ch):
            hs = [pltpu.async_copy(
                      table_hbm.at[idx_v.at[0, pl.ds((blk * ch + j) * 128, 128)]],
                      rows_v.at[j], sem)
                  for j in range(ch)]
            for h in hs:
                h.wait()
            pltpu.sync_copy(rows_v, out_hbm.at[pl.ds(ibase + blk * ch, ch)])

    return gk(table2, idxg)


def _gather(feat, idx):
    c = feat.shape[-1]
    table2 = feat.reshape(_B * _N, c)
    idxg = jnp.transpose(idx, (2, 0, 1)).reshape(32, 1, -1)
    nb = _sc_gather(table2, idxg)
    return nb.reshape(_K, _B * _N, c)


_RB_MLP = 512


def _edge_body(nb_ref, tab_ref, w0_ref, a0_ref, w1_ref, a1_ref,
               out_ref, *, cin, two):
    rep = tab_ref[0][:, :cin]
    a0 = a0_ref[...]
    acc = None
    for j in range(_K):
        nbj = nb_ref[j][:, :cin]
        f = jnp.concatenate([nbj - rep, rep], axis=1)
        h = _prelu(jnp.dot(f, w0_ref[...],
                           preferred_element_type=jnp.float32), a0)
        if two:
            h = _prelu(jnp.dot(h, w1_ref[...],
                               preferred_element_type=jnp.float32), a1_ref[...])
        acc = h if acc is None else jnp.maximum(acc, h)
    out_ref[0] = acc


def _edge_call(nb, tab, w0, a0, w1, a1, cin):
    ct = tab.shape[-1]
    two = w1 is not None
    a0r = a0.reshape(1, 1)
    if two:
        w1v, a1v = w1, a1.reshape(1, 1)
    else:
        w1v, a1v = jnp.zeros((64, 64), jnp.float32), jnp.zeros((1, 1), jnp.float32)
    grid = (_B, _N // _RB_MLP)
    full = lambda shape: pl.BlockSpec(shape, lambda b, r: tuple([0] * len(shape)))
    nblocks = _N // _RB_MLP
    return pl.pallas_call(
        functools.partial(_edge_body, cin=cin, two=two),
        grid=grid,
        in_specs=[
            pl.BlockSpec((_K, _RB_MLP, ct), lambda b, r: (0, b * nblocks + r, 0)),
            pl.BlockSpec((1, _RB_MLP, ct), lambda b, r: (b, r, 0)),
            full(w0.shape), full((1, 1)),
            full(w1v.shape), full((1, 1)),
        ],
        out_specs=pl.BlockSpec((1, _RB_MLP, 64), lambda b, r: (b, r, 0)),
        out_shape=jax.ShapeDtypeStruct((_B, _N, 64), jnp.float32),
        interpret=_INTERPRET,
    )(nb, tab, w0, a0r, w1v, a1v)


_CH_HEAD = 512


def _head_body(x1_ref, x2_ref, x3_ref, c4w_ref, c4a_ref, l1aw_ref, l1bw_ref,
               a1_ref, l2w_ref, a2_ref, l3w_ref, a3_ref, l4w_ref, a4_ref,
               out_ref):
    x4 = jnp.concatenate([x1_ref[0], x2_ref[0], x3_ref[0]], axis=1)
    mx = jnp.full((1, 1024), -jnp.inf, jnp.float32)
    for c in range(_N // _CH_HEAD):
        xc = x4[c * _CH_HEAD:(c + 1) * _CH_HEAD]
        z = _prelu(jnp.dot(xc, c4w_ref[...],
                           preferred_element_type=jnp.float32), c4a_ref[...])
        mx = jnp.maximum(mx, jnp.max(z, axis=0, keepdims=True))
    bias = jnp.dot(mx, l1bw_ref[...], preferred_element_type=jnp.float32)
    for c in range(_N // _CH_HEAD):
        xc = x4[c * _CH_HEAD:(c + 1) * _CH_HEAD]
        h = _prelu(jnp.dot(xc, l1aw_ref[...],
                           preferred_element_type=jnp.float32) + bias, a1_ref[...])
        h = _prelu(jnp.dot(h, l2w_ref[...],
                           preferred_element_type=jnp.float32), a2_ref[...])
        h = _prelu(jnp.dot(h, l3w_ref[...],
                           preferred_element_type=jnp.float32), a3_ref[...])
        h = _prelu(jnp.dot(h, l4w_ref[...],
                           preferred_element_type=jnp.float32), a4_ref[...])
        out_ref[0, c * _CH_HEAD:(c + 1) * _CH_HEAD, :] = h


def _head_call(x1, x2, x3, c4w, c4a, l1w, l1a, l2w, l2a, l3w, l3a, l4w, l4a):
    full = lambda shape: pl.BlockSpec(shape, lambda b: tuple([0] * len(shape)))
    args = [c4w, c4a.reshape(1, 1), l1w[:192], l1w[192:], l1a.reshape(1, 1),
            l2w, l2a.reshape(1, 1), l3w, l3a.reshape(1, 1), l4w,
            l4a.reshape(1, 1)]
    return pl.pallas_call(
        _head_body,
        grid=(_B,),
        in_specs=[
            pl.BlockSpec((1, _N, 64), lambda b: (b, 0, 0)),
            pl.BlockSpec((1, _N, 64), lambda b: (b, 0, 0)),
            pl.BlockSpec((1, _N, 64), lambda b: (b, 0, 0)),
        ] + [full(a.shape) for a in args],
        out_specs=pl.BlockSpec((1, _N, 4), lambda b: (b, 0, 0)),
        out_shape=jax.ShapeDtypeStruct((_B, _N, 4), jnp.float32),
        interpret=_INTERPRET,
    )(x1, x2, x3, *args)


def kernel(d, p, n, sdf_W1, sdf_b1, sdf_W2, sdf_b2, c1w0, c1a0, c1w1, c1a1,
           c2w0, c2a0, c2w1, c2a1, c3w0, c3a0, c4w, c4a, l1w, l1a, l2w, l2a,
           l3w, l3a, l4w, l4a):
    x = p + d * n
    sdf_res = jnp.tanh(x @ sdf_W1 + sdf_b1) @ sdf_W2 + sdf_b2
    sdf_grad = jax.lax.stop_gradient(jax.grad(lambda xq: jnp.sum(
        jnp.tanh(xq @ sdf_W1 + sdf_b1) @ sdf_W2 + sdf_b2))(x))
    li = jnp.concatenate(
        [x, n, sdf_res, sdf_grad, jnp.zeros((_B, _N, 6), jnp.float32)],
        axis=-1)
    idx1 = _knn_call(li)
    nb1 = _gather(li, idx1)
    x1 = _edge_call(nb1, li, c1w0, c1a0, c1w1, c1a1, cin=10)

    idx2 = _knn_call(x1)
    nb2 = _gather(x1, idx2)
    x2 = _edge_call(nb2, x1, c2w0, c2a0, c2w1, c2a1, cin=64)

    idx3 = _knn_call(x2)
    nb3 = _gather(x2, idx3)
    x3 = _edge_call(nb3, x2, c3w0, c3a0, None, None, cin=64)

    out4 = _head_call(x1, x2, x3, c4w, c4a, l1w, l1a, l2w, l2a, l3w, l3a,
                      l4w, l4a)
    return (out4[..., 0:1], out4[..., 1:2], out4[..., 2:3], out4[..., 3:4],
            x, sdf_res, sdf_grad)


# speedup vs baseline: 20.7159x; 1.1783x over previous
import functools

import jax
import jax.numpy as jnp
from jax import lax
from jax.experimental import pallas as pl
from jax.experimental.pallas import tpu as pltpu
from jax.experimental.pallas import tpu_sc as plsc

_K = 10
_N = 4096
_B = 2
_INTERPRET = False


def _prelu(x, a):
    return jnp.where(x >= 0, x, a * x)


_RB_KNN = 512


def _knn_body(rows_ref, all_ref, sqc_ref, sqr_ref, idx_ref):
    b = pl.program_id(0)
    xr = rows_ref[0]
    xa = all_ref[0]
    cross = lax.dot_general(xr, xa, (((1,), (1,)), ((), ())))
    d2 = sqc_ref[0] + sqr_ref[0] - 2.0 * cross
    colf = lax.broadcasted_iota(jnp.int32, d2.shape, 1).astype(jnp.float32)
    picks = []
    for _ in range(_K):
        m = jnp.min(d2, axis=1, keepdims=True)
        cand = jnp.where(d2 <= m, colf, jnp.float32(_N))
        pj = jnp.min(cand, axis=1, keepdims=True)
        picks.append(pj)
        d2 = jnp.where(colf == pj, jnp.inf, d2)
    idx_ref[0] = jnp.concatenate(picks, axis=1).astype(jnp.int32) + b * _N


def _knn_call(feat):
    c = feat.shape[-1]
    sq = jnp.sum(feat * feat, axis=-1)
    sqc = sq[:, :, None]
    sqr = sq[:, None, :]
    grid = (_B, _N // _RB_KNN)
    return pl.pallas_call(
        _knn_body,
        grid=grid,
        in_specs=[
            pl.BlockSpec((1, _RB_KNN, c), lambda b, r: (b, r, 0)),
            pl.BlockSpec((1, _N, c), lambda b, r: (b, 0, 0)),
            pl.BlockSpec((1, _RB_KNN, 1), lambda b, r: (b, r, 0)),
            pl.BlockSpec((1, 1, _N), lambda b, r: (b, 0, 0)),
        ],
        out_specs=pl.BlockSpec((1, _RB_KNN, _K), lambda b, r: (b, r, 0)),
        out_shape=jax.ShapeDtypeStruct((_B, _N, _K), jnp.int32),
        interpret=_INTERPRET,
    )(feat, feat, sqc, sqr)


def _sc_gather(table2, idxg):
    c = table2.shape[-1]
    nw, _, per_w = idxg.shape
    m = per_w // 128
    ch = 4
    mesh = plsc.VectorSubcoreMesh(core_axis_name="c", subcore_axis_name="s")

    @functools.partial(
        pl.kernel,
        out_type=jax.ShapeDtypeStruct((nw * m, 128, c), jnp.float32),
        mesh=mesh,
        scratch_types=[
            pltpu.VMEM((1, per_w), jnp.int32),
            pltpu.VMEM((ch, 128, c), jnp.float32),
            pltpu.SemaphoreType.DMA,
        ],
        compiler_params=pltpu.CompilerParams(use_tc_tiling_on_sc=False),
    )
    def gk(table_hbm, idx_hbm, out_hbm, idx_v, rows_v, sem):
        wid = lax.axis_index("s") * 2 + lax.axis_index("c")
        ibase = wid * m
        pltpu.sync_copy(idx_hbm.at[wid], idx_v)
        for blk in range(m // ch):
            hs = [pltpu.async_copy(
                      table_hbm.at[idx_v.at[0, pl.ds((blk * ch + j) * 128, 128)]],
                      rows_v.at[j], sem)
                  for j in range(ch)]
            for h in hs:
                h.wait()
            pltpu.sync_copy(rows_v, out_hbm.at[pl.ds(ibase + blk * ch, ch)])

    return gk(table2, idxg)


def _gather(feat, idx):
    c = feat.shape[-1]
    table2 = feat.reshape(_B * _N, c)
    idxg = jnp.transpose(idx, (2, 0, 1)).reshape(32, 1, -1)
    nb = _sc_gather(table2, idxg)
    return nb.reshape(_K, _B * _N, c)


_RB_MLP = 512


def _edge_body(nb_ref, tab_ref, w0_ref, a0_ref, w1_ref, a1_ref,
               out_ref, *, cin, two):
    rep = tab_ref[0][:, :cin]
    a0 = a0_ref[...]
    acc = None
    for j in range(_K):
        nbj = nb_ref[j][:, :cin]
        f = jnp.concatenate([nbj - rep, rep], axis=1)
        h = _prelu(jnp.dot(f, w0_ref[...],
                           preferred_element_type=jnp.float32), a0)
        if two:
            h = _prelu(jnp.dot(h, w1_ref[...],
                               preferred_element_type=jnp.float32), a1_ref[...])
        acc = h if acc is None else jnp.maximum(acc, h)
    out_ref[0] = acc


def _edge_call(nb, tab, w0, a0, w1, a1, cin):
    ct = tab.shape[-1]
    two = w1 is not None
    a0r = a0.reshape(1, 1)
    if two:
        w1v, a1v = w1, a1.reshape(1, 1)
    else:
        w1v, a1v = jnp.zeros((64, 64), jnp.float32), jnp.zeros((1, 1), jnp.float32)
    grid = (_B, _N // _RB_MLP)
    full = lambda shape: pl.BlockSpec(shape, lambda b, r: tuple([0] * len(shape)))
    nblocks = _N // _RB_MLP
    return pl.pallas_call(
        functools.partial(_edge_body, cin=cin, two=two),
        grid=grid,
        in_specs=[
            pl.BlockSpec((_K, _RB_MLP, ct), lambda b, r: (0, b * nblocks + r, 0)),
            pl.BlockSpec((1, _RB_MLP, ct), lambda b, r: (b, r, 0)),
            full(w0.shape), full((1, 1)),
            full(w1v.shape), full((1, 1)),
        ],
        out_specs=pl.BlockSpec((1, _RB_MLP, 64), lambda b, r: (b, r, 0)),
        out_shape=jax.ShapeDtypeStruct((_B, _N, 64), jnp.float32),
        interpret=_INTERPRET,
    )(nb, tab, w0, a0r, w1v, a1v)


_CH_HEAD = 512


def _head_body(x1_ref, x2_ref, x3_ref, c4w_ref, c4a_ref, l1aw_ref, l1bw_ref,
               a1_ref, l2w_ref, a2_ref, l3w_ref, a3_ref, l4w_ref, a4_ref,
               out_ref):
    x4 = jnp.concatenate([x1_ref[0], x2_ref[0], x3_ref[0]], axis=1)
    mx = jnp.full((1, 1024), -jnp.inf, jnp.float32)
    for c in range(_N // _CH_HEAD):
        xc = x4[c * _CH_HEAD:(c + 1) * _CH_HEAD]
        z = _prelu(jnp.dot(xc, c4w_ref[...],
                           preferred_element_type=jnp.float32), c4a_ref[...])
        mx = jnp.maximum(mx, jnp.max(z, axis=0, keepdims=True))
    bias = jnp.dot(mx, l1bw_ref[...], preferred_element_type=jnp.float32)
    for c in range(_N // _CH_HEAD):
        xc = x4[c * _CH_HEAD:(c + 1) * _CH_HEAD]
        h = _prelu(jnp.dot(xc, l1aw_ref[...],
                           preferred_element_type=jnp.float32) + bias, a1_ref[...])
        h = _prelu(jnp.dot(h, l2w_ref[...],
                           preferred_element_type=jnp.float32), a2_ref[...])
        h = _prelu(jnp.dot(h, l3w_ref[...],
                           preferred_element_type=jnp.float32), a3_ref[...])
        h = _prelu(jnp.dot(h, l4w_ref[...],
                           preferred_element_type=jnp.float32), a4_ref[...])
        out_ref[0, c * _CH_HEAD:(c + 1) * _CH_HEAD, :] = h


def _head_call(x1, x2, x3, c4w, c4a, l1w, l1a, l2w, l2a, l3w, l3a, l4w, l4a):
    full = lambda shape: pl.BlockSpec(shape, lambda b: tuple([0] * len(shape)))
    args = [c4w, c4a.reshape(1, 1), l1w[:192], l1w[192:], l1a.reshape(1, 1),
            l2w, l2a.reshape(1, 1), l3w, l3a.reshape(1, 1), l4w,
            l4a.reshape(1, 1)]
    return pl.pallas_call(
        _head_body,
        grid=(_B,),
        in_specs=[
            pl.BlockSpec((1, _N, 64), lambda b: (b, 0, 0)),
            pl.BlockSpec((1, _N, 64), lambda b: (b, 0, 0)),
            pl.BlockSpec((1, _N, 64), lambda b: (b, 0, 0)),
        ] + [full(a.shape) for a in args],
        out_specs=pl.BlockSpec((1, _N, 4), lambda b: (b, 0, 0)),
        out_shape=jax.ShapeDtypeStruct((_B, _N, 4), jnp.float32),
        interpret=_INTERPRET,
    )(x1, x2, x3, *args)


def kernel(d, p, n, sdf_W1, sdf_b1, sdf_W2, sdf_b2, c1w0, c1a0, c1w1, c1a1,
           c2w0, c2a0, c2w1, c2a1, c3w0, c3a0, c4w, c4a, l1w, l1a, l2w, l2a,
           l3w, l3a, l4w, l4a):
    x = p + d * n
    sdf_res = jnp.tanh(x @ sdf_W1 + sdf_b1) @ sdf_W2 + sdf_b2
    sdf_grad = jax.lax.stop_gradient(jax.grad(lambda xq: jnp.sum(
        jnp.tanh(xq @ sdf_W1 + sdf_b1) @ sdf_W2 + sdf_b2))(x))
    li = jnp.concatenate(
        [x, n, sdf_res, sdf_grad, jnp.zeros((_B, _N, 6), jnp.float32)],
        axis=-1)
    idx1 = _knn_call(li)
    nb1 = _gather(li, idx1)
    x1 = _edge_call(nb1, li, c1w0, c1a0, c1w1, c1a1, cin=10)

    idx2 = _knn_call(x1)
    nb2 = _gather(x1, idx2)
    x2 = _edge_call(nb2, x1, c2w0, c2a0, c2w1, c2a1, cin=64)

    idx3 = _knn_call(x2)
    nb3 = _gather(x2, idx3)
    x3 = _edge_call(nb3, x2, c3w0, c3a0, None, None, cin=64)

    out4 = _head_call(x1, x2, x3, c4w, c4a, l1w, l1a, l2w, l2a, l3w, l3a,
                      l4w, l4a)
    return (out4[..., 0:1], out4[..., 1:2], out4[..., 2:3], out4[..., 3:4],
            x, sdf_res, sdf_grad)
